# baseline (speedup 1.0000x reference)
import functools

import jax
import jax.numpy as jnp
import numpy as np
from jax import lax
from jax.experimental import pallas as pl
from jax.experimental.pallas import tpu as pltpu
from jax.experimental.pallas import tpu_sc as plsc

ROWS = 64
COLS = 1_000_000
BC = 8192
CW = 512
NCHUNK = BC // CW

SC_BLOCKS = 36
SCCOLS = SC_BLOCKS * BC
GRID1 = (COLS - SCCOLS + BC - 1) // BC

NC, NS = 2, 16
NW = NC * NS
CHUNK = 8192
VECS = CHUNK // 16
E_CHUNKS = ROWS * SCCOLS // (NW * CHUNK)

_KS1 = np.uint32(42)
_KS2 = np.uint32(42 ^ 0x1BD11BDA)
_ROT0 = (13, 15, 26, 6)
_ROT1 = (17, 29, 16, 24)
_IMAX = np.int32(np.iinfo(np.int32).max)

_LN2 = np.float32(0.6931471805599453)
_GC = np.float32(0.36651292058166432)


def _rounds(x0, x1, rots):
    for r in rots:
        x0 = x0 + x1
        x1 = (x1 << r) | (x1 >> (32 - r))
        x1 = x1 ^ x0
    return x0, x1


def _threefry_bits(x1):
    x0 = x1
    r = _ROT0[0]
    x1 = (x1 << r) | (x1 >> (32 - r))
    x1 = x1 ^ x0
    x0, x1 = _rounds(x0, x1, _ROT0[1:])
    x0 = x0 + _KS1
    x1 = x1 + np.uint32(_KS2 + np.uint32(1))
    x0, x1 = _rounds(x0, x1, _ROT1)
    x0 = x0 + _KS2
    x1 = x1 + np.uint32(2)
    x0, x1 = _rounds(x0, x1, _ROT0)
    x1 = x1 + np.uint32(_KS1 + np.uint32(3))
    x0, x1 = _rounds(x0, x1, _ROT1)
    x0 = x0 + _KS1
    x1 = x1 + np.uint32(_KS2 + np.uint32(4))
    x0, x1 = _rounds(x0, x1, _ROT0)
    x0 = x0 + _KS2
    x1 = x1 + np.uint32(5)
    return x0 ^ x1


def _bits_to_f(bits):
    fb = (bits >> np.uint32(9)) | np.uint32(0x3F800000)
    return lax.bitcast_convert_type(fb, jnp.float32) - np.float32(1.0)


def _f_to_gumbel(f):
    t = -jnp.log2(f)
    return _GC - _LN2 * jnp.log2(t)


def _sc_kernel(out_ref, buf0, buf1, sem0, sem1):
    w = lax.axis_index("s") * NC + lax.axis_index("c")
    gc0 = w * E_CHUNKS
    iota16 = lax.iota(jnp.int32, 16)

    def chunk_into(buf, gc):
        row = gc // SC_BLOCKS
        cc = gc - row * SC_BLOCKS
        base_i = row * COLS + cc * CHUNK + 42

        def vbody(v, _):
            x1 = (iota16 + (base_i + v * 16)).astype(jnp.uint32)
            buf[pl.ds(v * 16, 16)] = _bits_to_f(_threefry_bits(x1))
            return 0

        lax.fori_loop(0, VECS, vbody, 0, unroll=8)

    def copy(buf, gc, sem):
        row = gc // SC_BLOCKS
        cc = gc - row * SC_BLOCKS
        return pltpu.make_async_copy(
            buf, out_ref.at[row, pl.ds(cc * CHUNK, CHUNK)], sem)

    def outer(j, _):
        gc = gc0 + j * 2

        @pl.when(j > 0)
        def _():
            copy(buf0, gc - 2, sem0).wait()

        chunk_into(buf0, gc)
        copy(buf0, gc, sem0).start()

        @pl.when(j > 0)
        def _():
            copy(buf1, gc - 1, sem1).wait()

        chunk_into(buf1, gc + 1)
        copy(buf1, gc + 1, sem1).start()
        return 0

    lax.fori_loop(0, E_CHUNKS // 2, outer, 0)
    copy(buf0, gc0 + E_CHUNKS - 2, sem0).wait()
    copy(buf1, gc0 + E_CHUNKS - 1, sem1).wait()


@functools.partial(
    pl.kernel,
    mesh=plsc.VectorSubcoreMesh(core_axis_name="c", subcore_axis_name="s"),
    out_type=jax.ShapeDtypeStruct((ROWS, SCCOLS), jnp.float32),
    scratch_types=[
        pltpu.VMEM((CHUNK,), jnp.float32),
        pltpu.VMEM((CHUNK,), jnp.float32),
        pltpu.SemaphoreType.DMA,
        pltpu.SemaphoreType.DMA,
    ],
)
def _sc_uniforms(out_ref, buf0, buf1, sem0, sem1):
    _sc_kernel(out_ref, buf0, buf1, sem0, sem1)


def _tc1_kernel(x_ref, omax_ref, oidx_ref, run_ref, idx_ref):
    k = pl.program_id(0)

    @pl.when(k == 0)
    def _init():
        run_ref[...] = jnp.full((ROWS, BC), -jnp.inf, jnp.float32)
        idx_ref[...] = jnp.zeros((ROWS, BC), jnp.int32)

    base = SCCOLS + k * BC
    rows_off = (lax.broadcasted_iota(jnp.int32, (ROWS, CW), 0)
                * COLS).astype(jnp.uint32)
    lane = lax.broadcasted_iota(jnp.int32, (ROWS, CW), 1)
    for j in range(NCHUNK):
        sl = slice(j * CW, (j + 1) * CW)
        cols = lane + (base + j * CW)
        x1 = rows_off + (cols + 42).astype(jnp.uint32)
        g = _f_to_gumbel(_bits_to_f(_threefry_bits(x1)))
        val = jnp.where(cols < COLS, x_ref[:, sl] + g, -jnp.inf)

        run = run_ref[:, sl]
        m = val > run
        run_ref[:, sl] = jnp.where(m, val, run)
        idx_ref[:, sl] = jnp.where(m, cols, idx_ref[:, sl])

    @pl.when(k == GRID1 - 1)
    def _finalize():
        run = run_ref[...]
        idx = idx_ref[...]
        mx = jnp.max(run, axis=1, keepdims=True)
        cand = jnp.where(run == mx, idx, _IMAX)
        ix = jnp.min(cand, axis=1, keepdims=True)
        omax_ref[...] = jnp.broadcast_to(mx, (ROWS, 128))
        oidx_ref[...] = jnp.broadcast_to(ix, (ROWS, 128))


def _tc2_kernel(x_ref, f_ref, pmax_ref, pidx_ref, o_ref, run_ref, idx_ref):
    k = pl.program_id(0)

    @pl.when(k == 0)
    def _init():
        run_ref[...] = jnp.full((ROWS, BC), -jnp.inf, jnp.float32)
        idx_ref[...] = jnp.zeros((ROWS, BC), jnp.int32)

    base = k * BC
    cw2 = 512
    lane = lax.broadcasted_iota(jnp.int32, (ROWS, cw2), 1)
    for j in range(BC // cw2):
        sl = slice(j * cw2, (j + 1) * cw2)
        cols = lane + (base + j * cw2)
        val = x_ref[:, sl] + _f_to_gumbel(f_ref[:, sl])

        run = run_ref[:, sl]
        m = val > run
        run_ref[:, sl] = jnp.where(m, val, run)
        idx_ref[:, sl] = jnp.where(m, cols, idx_ref[:, sl])

    @pl.when(k == SC_BLOCKS - 1)
    def _finalize():
        run = run_ref[...]
        idx = idx_ref[...]
        m2 = jnp.max(run, axis=1, keepdims=True)
        pm = pmax_ref[:, 0:1]
        mx = jnp.maximum(m2, pm)
        c2 = jnp.min(jnp.where(run == mx, idx, _IMAX), axis=1, keepdims=True)
        c1 = jnp.where(pm == mx, pidx_ref[:, 0:1], _IMAX)
        o_ref[...] = jnp.minimum(c2, c1)


def kernel(x):
    f = _sc_uniforms()
    pmax, pidx = pl.pallas_call(
        _tc1_kernel,
        grid=(GRID1,),
        in_specs=[pl.BlockSpec((ROWS, BC), lambda k: (0, k + SC_BLOCKS))],
        out_specs=[
            pl.BlockSpec((ROWS, 128), lambda k: (0, 0)),
            pl.BlockSpec((ROWS, 128), lambda k: (0, 0)),
        ],
        out_shape=[
            jax.ShapeDtypeStruct((ROWS, 128), jnp.float32),
            jax.ShapeDtypeStruct((ROWS, 128), jnp.int32),
        ],
        scratch_shapes=[
            pltpu.VMEM((ROWS, BC), jnp.float32),
            pltpu.VMEM((ROWS, BC), jnp.int32),
        ],
    )(x)
    out = pl.pallas_call(
        _tc2_kernel,
        grid=(SC_BLOCKS,),
        in_specs=[
            pl.BlockSpec((ROWS, BC), lambda k: (0, k)),
            pl.BlockSpec((ROWS, BC), lambda k: (0, k)),
            pl.BlockSpec((ROWS, 128), lambda k: (0, 0)),
            pl.BlockSpec((ROWS, 128), lambda k: (0, 0)),
        ],
        out_specs=pl.BlockSpec((ROWS, 1), lambda k: (0, 0)),
        out_shape=jax.ShapeDtypeStruct((ROWS, 1), jnp.int32),
        scratch_shapes=[
            pltpu.VMEM((ROWS, BC), jnp.float32),
            pltpu.VMEM((ROWS, BC), jnp.int32),
        ],
    )(x, f, pmax, pidx)
    return out[:, 0]


# speedup vs baseline: 1.0267x; 1.0267x over previous
import functools

import jax
import jax.numpy as jnp
import numpy as np
from jax import lax
from jax.experimental import pallas as pl
from jax.experimental.pallas import tpu as pltpu
from jax.experimental.pallas import tpu_sc as plsc

ROWS = 64
COLS = 1_000_000
BC = 8192
CW = 512
NCHUNK = BC // CW

SC_BLOCKS = 35
SCCOLS = SC_BLOCKS * BC
GRID1 = (COLS - SCCOLS + BC - 1) // BC

NC, NS = 2, 16
NW = NC * NS
CHUNK = 8192
VECS = CHUNK // 16
E_CHUNKS = ROWS * SCCOLS // (NW * CHUNK)

_KS1 = np.uint32(42)
_KS2 = np.uint32(42 ^ 0x1BD11BDA)
_ROT0 = (13, 15, 26, 6)
_ROT1 = (17, 29, 16, 24)
_IMAX = np.int32(np.iinfo(np.int32).max)

_LN2 = np.float32(0.6931471805599453)
_GC = np.float32(0.36651292058166432)


def _rounds(x0, x1, rots):
    for r in rots:
        x0 = x0 + x1
        x1 = (x1 << r) | (x1 >> (32 - r))
        x1 = x1 ^ x0
    return x0, x1


def _threefry_bits(x1):
    x0 = x1
    r = _ROT0[0]
    x1 = (x1 << r) | (x1 >> (32 - r))
    x1 = x1 ^ x0
    x0, x1 = _rounds(x0, x1, _ROT0[1:])
    x0 = x0 + _KS1
    x1 = x1 + np.uint32(_KS2 + np.uint32(1))
    x0, x1 = _rounds(x0, x1, _ROT1)
    x0 = x0 + _KS2
    x1 = x1 + np.uint32(2)
    x0, x1 = _rounds(x0, x1, _ROT0)
    x1 = x1 + np.uint32(_KS1 + np.uint32(3))
    x0, x1 = _rounds(x0, x1, _ROT1)
    x0 = x0 + _KS1
    x1 = x1 + np.uint32(_KS2 + np.uint32(4))
    x0, x1 = _rounds(x0, x1, _ROT0)
    x0 = x0 + _KS2
    x1 = x1 + np.uint32(5)
    return x0 ^ x1


def _bits_to_f(bits):
    fb = (bits >> np.uint32(9)) | np.uint32(0x3F800000)
    return lax.bitcast_convert_type(fb, jnp.float32) - np.float32(1.0)


def _f_to_gumbel(f):
    t = -jnp.log2(f)
    return _GC - _LN2 * jnp.log2(t)


def _sc_kernel(out_ref, buf0, buf1, sem0, sem1):
    w = lax.axis_index("s") * NC + lax.axis_index("c")
    gc0 = w * E_CHUNKS
    iota16 = lax.iota(jnp.int32, 16)

    def chunk_into(buf, gc):
        row = gc // SC_BLOCKS
        cc = gc - row * SC_BLOCKS
        base_i = row * COLS + cc * CHUNK + 42

        def vbody(v, _):
            x1 = (iota16 + (base_i + v * 16)).astype(jnp.uint32)
            buf[pl.ds(v * 16, 16)] = _bits_to_f(_threefry_bits(x1))
            return 0

        lax.fori_loop(0, VECS, vbody, 0, unroll=8)

    def copy(buf, gc, sem):
        row = gc // SC_BLOCKS
        cc = gc - row * SC_BLOCKS
        return pltpu.make_async_copy(
            buf, out_ref.at[row, pl.ds(cc * CHUNK, CHUNK)], sem)

    def outer(j, _):
        gc = gc0 + j * 2

        @pl.when(j > 0)
        def _():
            copy(buf0, gc - 2, sem0).wait()

        chunk_into(buf0, gc)
        copy(buf0, gc, sem0).start()

        @pl.when(j > 0)
        def _():
            copy(buf1, gc - 1, sem1).wait()

        chunk_into(buf1, gc + 1)
        copy(buf1, gc + 1, sem1).start()
        return 0

    lax.fori_loop(0, E_CHUNKS // 2, outer, 0)
    copy(buf0, gc0 + E_CHUNKS - 2, sem0).wait()
    copy(buf1, gc0 + E_CHUNKS - 1, sem1).wait()


@functools.partial(
    pl.kernel,
    mesh=plsc.VectorSubcoreMesh(core_axis_name="c", subcore_axis_name="s"),
    out_type=jax.ShapeDtypeStruct((ROWS, SCCOLS), jnp.float32),
    scratch_types=[
        pltpu.VMEM((CHUNK,), jnp.float32),
        pltpu.VMEM((CHUNK,), jnp.float32),
        pltpu.SemaphoreType.DMA,
        pltpu.SemaphoreType.DMA,
    ],
)
def _sc_uniforms(out_ref, buf0, buf1, sem0, sem1):
    _sc_kernel(out_ref, buf0, buf1, sem0, sem1)


def _tc1_kernel(x_ref, omax_ref, oidx_ref, run_ref, idx_ref):
    k = pl.program_id(0)

    @pl.when(k == 0)
    def _init():
        run_ref[...] = jnp.full((ROWS, BC), -jnp.inf, jnp.float32)
        idx_ref[...] = jnp.zeros((ROWS, BC), jnp.int32)

    base = SCCOLS + k * BC
    rows_off = (lax.broadcasted_iota(jnp.int32, (ROWS, CW), 0)
                * COLS).astype(jnp.uint32)
    lane = lax.broadcasted_iota(jnp.int32, (ROWS, CW), 1)
    for j in range(NCHUNK):
        sl = slice(j * CW, (j + 1) * CW)
        cols = lane + (base + j * CW)
        x1 = rows_off + (cols + 42).astype(jnp.uint32)
        g = _f_to_gumbel(_bits_to_f(_threefry_bits(x1)))
        val = jnp.where(cols < COLS, x_ref[:, sl] + g, -jnp.inf)

        run = run_ref[:, sl]
        m = val > run
        run_ref[:, sl] = jnp.where(m, val, run)
        idx_ref[:, sl] = jnp.where(m, cols, idx_ref[:, sl])

    @pl.when(k == GRID1 - 1)
    def _finalize():
        run = run_ref[...]
        idx = idx_ref[...]
        mx = jnp.max(run, axis=1, keepdims=True)
        cand = jnp.where(run == mx, idx, _IMAX)
        ix = jnp.min(cand, axis=1, keepdims=True)
        omax_ref[...] = jnp.broadcast_to(mx, (ROWS, 128))
        oidx_ref[...] = jnp.broadcast_to(ix, (ROWS, 128))


def _tc2_kernel(x_ref, f_ref, pmax_ref, pidx_ref, o_ref, run_ref, idx_ref):
    k = pl.program_id(0)

    @pl.when(k == 0)
    def _init():
        run_ref[...] = jnp.full((ROWS, BC), -jnp.inf, jnp.float32)
        idx_ref[...] = jnp.zeros((ROWS, BC), jnp.int32)

    base = k * BC
    cw2 = 512
    lane = lax.broadcasted_iota(jnp.int32, (ROWS, cw2), 1)
    for j in range(BC // cw2):
        sl = slice(j * cw2, (j + 1) * cw2)
        cols = lane + (base + j * cw2)
        val = x_ref[:, sl] + _f_to_gumbel(f_ref[:, sl])

        run = run_ref[:, sl]
        m = val > run
        run_ref[:, sl] = jnp.where(m, val, run)
        idx_ref[:, sl] = jnp.where(m, cols, idx_ref[:, sl])

    @pl.when(k == SC_BLOCKS - 1)
    def _finalize():
        run = run_ref[...]
        idx = idx_ref[...]
        m2 = jnp.max(run, axis=1, keepdims=True)
        pm = pmax_ref[:, 0:1]
        mx = jnp.maximum(m2, pm)
        c2 = jnp.min(jnp.where(run == mx, idx, _IMAX), axis=1, keepdims=True)
        c1 = jnp.where(pm == mx, pidx_ref[:, 0:1], _IMAX)
        o_ref[...] = jnp.minimum(c2, c1)


def kernel(x):
    f = _sc_uniforms()
    pmax, pidx = pl.pallas_call(
        _tc1_kernel,
        grid=(GRID1,),
        in_specs=[pl.BlockSpec((ROWS, BC), lambda k: (0, k + SC_BLOCKS))],
        out_specs=[
            pl.BlockSpec((ROWS, 128), lambda k: (0, 0)),
            pl.BlockSpec((ROWS, 128), lambda k: (0, 0)),
        ],
        out_shape=[
            jax.ShapeDtypeStruct((ROWS, 128), jnp.float32),
            jax.ShapeDtypeStruct((ROWS, 128), jnp.int32),
        ],
        scratch_shapes=[
            pltpu.VMEM((ROWS, BC), jnp.float32),
            pltpu.VMEM((ROWS, BC), jnp.int32),
        ],
    )(x)
    out = pl.pallas_call(
        _tc2_kernel,
        grid=(SC_BLOCKS,),
        in_specs=[
            pl.BlockSpec((ROWS, BC), lambda k: (0, k)),
            pl.BlockSpec((ROWS, BC), lambda k: (0, k)),
            pl.BlockSpec((ROWS, 128), lambda k: (0, 0)),
            pl.BlockSpec((ROWS, 128), lambda k: (0, 0)),
        ],
        out_specs=pl.BlockSpec((ROWS, 1), lambda k: (0, 0)),
        out_shape=jax.ShapeDtypeStruct((ROWS, 1), jnp.int32),
        scratch_shapes=[
            pltpu.VMEM((ROWS, BC), jnp.float32),
            pltpu.VMEM((ROWS, BC), jnp.int32),
        ],
    )(x, f, pmax, pidx)
    return out[:, 0]


# speedup vs baseline: 1.0305x; 1.0036x over previous
import functools

import jax
import jax.numpy as jnp
import numpy as np
from jax import lax
from jax.experimental import pallas as pl
from jax.experimental.pallas import tpu as pltpu
from jax.experimental.pallas import tpu_sc as plsc

ROWS = 64
COLS = 1_000_000
BC = 8192
CW = 512
NCHUNK = BC // CW

SC_BLOCKS = 35
SCCOLS = SC_BLOCKS * BC
GRID1 = (COLS - SCCOLS + BC - 1) // BC

NC, NS = 2, 16
NW = NC * NS
CHUNK = 8192
VECS = CHUNK // 16
E_CHUNKS = ROWS * SCCOLS // (NW * CHUNK)

_KS1 = np.uint32(42)
_KS2 = np.uint32(42 ^ 0x1BD11BDA)
_ROT0 = (13, 15, 26, 6)
_ROT1 = (17, 29, 16, 24)
_IMAX = np.int32(np.iinfo(np.int32).max)

_LN2 = np.float32(0.6931471805599453)
_GC = np.float32(0.36651292058166432)


def _rounds(x0, x1, rots):
    for r in rots:
        x0 = x0 + x1
        x1 = (x1 << r) | (x1 >> (32 - r))
        x1 = x1 ^ x0
    return x0, x1


def _threefry_bits(x1):
    x0 = x1
    r = _ROT0[0]
    x1 = (x1 << r) | (x1 >> (32 - r))
    x1 = x1 ^ x0
    x0, x1 = _rounds(x0, x1, _ROT0[1:])
    x0 = x0 + _KS1
    x1 = x1 + np.uint32(_KS2 + np.uint32(1))
    x0, x1 = _rounds(x0, x1, _ROT1)
    x0 = x0 + _KS2
    x1 = x1 + np.uint32(2)
    x0, x1 = _rounds(x0, x1, _ROT0)
    x1 = x1 + np.uint32(_KS1 + np.uint32(3))
    x0, x1 = _rounds(x0, x1, _ROT1)
    x0 = x0 + _KS1
    x1 = x1 + np.uint32(_KS2 + np.uint32(4))
    x0, x1 = _rounds(x0, x1, _ROT0)
    x0 = x0 + _KS2
    x1 = x1 + np.uint32(5)
    return x0 ^ x1


def _bits_to_f(bits):
    fb = (bits >> np.uint32(9)) | np.uint32(0x3F800000)
    return lax.bitcast_convert_type(fb, jnp.float32) - np.float32(1.0)


def _f_to_gumbel(f):
    t = -jnp.log2(f)
    return _GC - _LN2 * jnp.log2(t)


def _sc_kernel(out_ref, buf0, buf1, sem0, sem1):
    w = lax.axis_index("s") * NC + lax.axis_index("c")
    gc0 = w * E_CHUNKS
    iota16 = lax.iota(jnp.int32, 16)

    def chunk_into(buf, gc):
        row = gc // SC_BLOCKS
        cc = gc - row * SC_BLOCKS
        base_i = row * COLS + cc * CHUNK + 42

        def vbody(v, _):
            x1 = (iota16 + (base_i + v * 16)).astype(jnp.uint32)
            buf[pl.ds(v * 16, 16)] = _threefry_bits(x1)
            return 0

        lax.fori_loop(0, VECS, vbody, 0, unroll=8)

    def copy(buf, gc, sem):
        row = gc // SC_BLOCKS
        cc = gc - row * SC_BLOCKS
        return pltpu.make_async_copy(
            buf, out_ref.at[row, pl.ds(cc * CHUNK, CHUNK)], sem)

    def outer(j, _):
        gc = gc0 + j * 2

        @pl.when(j > 0)
        def _():
            copy(buf0, gc - 2, sem0).wait()

        chunk_into(buf0, gc)
        copy(buf0, gc, sem0).start()

        @pl.when(j > 0)
        def _():
            copy(buf1, gc - 1, sem1).wait()

        chunk_into(buf1, gc + 1)
        copy(buf1, gc + 1, sem1).start()
        return 0

    lax.fori_loop(0, E_CHUNKS // 2, outer, 0)
    copy(buf0, gc0 + E_CHUNKS - 2, sem0).wait()
    copy(buf1, gc0 + E_CHUNKS - 1, sem1).wait()


@functools.partial(
    pl.kernel,
    mesh=plsc.VectorSubcoreMesh(core_axis_name="c", subcore_axis_name="s"),
    out_type=jax.ShapeDtypeStruct((ROWS, SCCOLS), jnp.uint32),
    scratch_types=[
        pltpu.VMEM((CHUNK,), jnp.uint32),
        pltpu.VMEM((CHUNK,), jnp.uint32),
        pltpu.SemaphoreType.DMA,
        pltpu.SemaphoreType.DMA,
    ],
)
def _sc_uniforms(out_ref, buf0, buf1, sem0, sem1):
    _sc_kernel(out_ref, buf0, buf1, sem0, sem1)


def _tc1_kernel(x_ref, omax_ref, oidx_ref, run_ref, idx_ref):
    k = pl.program_id(0)

    @pl.when(k == 0)
    def _init():
        run_ref[...] = jnp.full((ROWS, BC), -jnp.inf, jnp.float32)
        idx_ref[...] = jnp.zeros((ROWS, BC), jnp.int32)

    base = SCCOLS + k * BC
    rows_off = (lax.broadcasted_iota(jnp.int32, (ROWS, CW), 0)
                * COLS).astype(jnp.uint32)
    lane = lax.broadcasted_iota(jnp.int32, (ROWS, CW), 1)
    for j in range(NCHUNK):
        sl = slice(j * CW, (j + 1) * CW)
        cols = lane + (base + j * CW)
        x1 = rows_off + (cols + 42).astype(jnp.uint32)
        g = _f_to_gumbel(_bits_to_f(_threefry_bits(x1)))
        val = jnp.where(cols < COLS, x_ref[:, sl] + g, -jnp.inf)

        run = run_ref[:, sl]
        m = val > run
        run_ref[:, sl] = jnp.where(m, val, run)
        idx_ref[:, sl] = jnp.where(m, cols, idx_ref[:, sl])

    @pl.when(k == GRID1 - 1)
    def _finalize():
        run = run_ref[...]
        idx = idx_ref[...]
        mx = jnp.max(run, axis=1, keepdims=True)
        cand = jnp.where(run == mx, idx, _IMAX)
        ix = jnp.min(cand, axis=1, keepdims=True)
        omax_ref[...] = jnp.broadcast_to(mx, (ROWS, 128))
        oidx_ref[...] = jnp.broadcast_to(ix, (ROWS, 128))


def _tc2_kernel(x_ref, f_ref, pmax_ref, pidx_ref, o_ref, run_ref, idx_ref):
    k = pl.program_id(0)

    @pl.when(k == 0)
    def _init():
        run_ref[...] = jnp.full((ROWS, BC), -jnp.inf, jnp.float32)
        idx_ref[...] = jnp.zeros((ROWS, BC), jnp.int32)

    base = k * BC
    cw2 = 512
    lane = lax.broadcasted_iota(jnp.int32, (ROWS, cw2), 1)
    for j in range(BC // cw2):
        sl = slice(j * cw2, (j + 1) * cw2)
        cols = lane + (base + j * cw2)
        val = x_ref[:, sl] + _f_to_gumbel(_bits_to_f(f_ref[:, sl]))

        run = run_ref[:, sl]
        m = val > run
        run_ref[:, sl] = jnp.where(m, val, run)
        idx_ref[:, sl] = jnp.where(m, cols, idx_ref[:, sl])

    @pl.when(k == SC_BLOCKS - 1)
    def _finalize():
        run = run_ref[...]
        idx = idx_ref[...]
        m2 = jnp.max(run, axis=1, keepdims=True)
        pm = pmax_ref[:, 0:1]
        mx = jnp.maximum(m2, pm)
        c2 = jnp.min(jnp.where(run == mx, idx, _IMAX), axis=1, keepdims=True)
        c1 = jnp.where(pm == mx, pidx_ref[:, 0:1], _IMAX)
        o_ref[...] = jnp.minimum(c2, c1)


def kernel(x):
    f = _sc_uniforms()
    pmax, pidx = pl.pallas_call(
        _tc1_kernel,
        grid=(GRID1,),
        in_specs=[pl.BlockSpec((ROWS, BC), lambda k: (0, k + SC_BLOCKS))],
        out_specs=[
            pl.BlockSpec((ROWS, 128), lambda k: (0, 0)),
            pl.BlockSpec((ROWS, 128), lambda k: (0, 0)),
        ],
        out_shape=[
            jax.ShapeDtypeStruct((ROWS, 128), jnp.float32),
            jax.ShapeDtypeStruct((ROWS, 128), jnp.int32),
        ],
        scratch_shapes=[
            pltpu.VMEM((ROWS, BC), jnp.float32),
            pltpu.VMEM((ROWS, BC), jnp.int32),
        ],
    )(x)
    out = pl.pallas_call(
        _tc2_kernel,
        grid=(SC_BLOCKS,),
        in_specs=[
            pl.BlockSpec((ROWS, BC), lambda k: (0, k)),
            pl.BlockSpec((ROWS, BC), lambda k: (0, k)),
            pl.BlockSpec((ROWS, 128), lambda k: (0, 0)),
            pl.BlockSpec((ROWS, 128), lambda k: (0, 0)),
        ],
        out_specs=pl.BlockSpec((ROWS, 1), lambda k: (0, 0)),
        out_shape=jax.ShapeDtypeStruct((ROWS, 1), jnp.int32),
        scratch_shapes=[
            pltpu.VMEM((ROWS, BC), jnp.float32),
            pltpu.VMEM((ROWS, BC), jnp.int32),
        ],
    )(x, f, pmax, pidx)
    return out[:, 0]
